# baseline (speedup 1.0000x reference)
import jax
import jax.numpy as jnp
from jax import lax
from jax.experimental import pallas as pl
from jax.experimental.pallas import tpu as pltpu
from jax.experimental.pallas import tpu_sc as plsc

NC, NS, LANES = 2, 16, 16
NW = NC * NS
C = 32
R8 = 1024
EPS = 1e-12


def _prep_rel_body(rel3_ref, ak0_ref, ak1_ref, nrm_ref, rl0_ref, rl1_ref):
    r = rel3_ref[...]
    nrm_ref[...] = jnp.sqrt(jnp.sum(r * r, axis=2))
    rl0_ref[...] = jnp.sum(r * ak0_ref[...], axis=2)
    rl1_ref[...] = jnp.sum(r * ak1_ref[...], axis=2)


def _tanh_body(x_ref, o_ref):
    o_ref[...] = jnp.tanh(x_ref[...])


def _dmat_body(f_ref, r_ref, o_ref):
    o_ref[...] = lax.dot_general(
        f_ref[...], r_ref[...], (((1,), (1,)), ((), ())),
        preferred_element_type=jnp.float32)


def _make_sc_depth(d, rpt, slot, n_pad):
    mesh = plsc.VectorSubcoreMesh(core_axis_name="c", subcore_axis_name="s",
                                  num_cores=NC, num_subcores=NS)
    nk = d // LANES
    ng = C // LANES

    def body(feats_hbm, rel_hbm, dflat_hbm, rowp, colp, rcp, rvalp, nrm_hbm,
             rl_hbm, binfo_hbm, zrows_hbm, out_hbm, s_hbm, bbuf, rowb, colb,
             rcb, rvb, ngb, rgb, sgb, dbuf, estage, gidxb, didxb, sloc, fbuf,
             rbuf, acc, s_sh, semS, semG, semA, sem2):
        cidx = lax.axis_index("c")
        sidx = lax.axis_index("s")
        w = cidx * NS + sidx
        lane = lax.iota(jnp.int32, LANES)

        pltpu.sync_copy(binfo_hbm.at[w], bbuf)
        bv = bbuf[...]
        e_lo, e_hi, a0, nch = bv[0], bv[1], bv[2], bv[3]
        nche = ((nch + 1) // 2) * 2
        r0 = pl.multiple_of(w * rpt, 8)
        zf = jnp.zeros((LANES,), jnp.float32)

        cp_z = pltpu.async_copy(zrows_hbm, acc, sem2)
        for i in range(slot // LANES):
            sloc[pl.ds(i * LANES, LANES)] = zf
        sbase = pl.multiple_of(sidx * slot, 8)
        pltpu.sync_copy(sloc, s_sh.at[pl.ds(sbase, slot)])
        cp_z.wait()

        def chunk_start(i):
            return pl.multiple_of(a0 + i * C, 8)

        def smalls_copies(i, p):
            st = chunk_start(i)
            return (
                pltpu.make_async_copy(rowp.at[pl.ds(st, C)], rowb.at[p],
                                      semS.at[p]),
                pltpu.make_async_copy(colp.at[pl.ds(st, C)], colb.at[p],
                                      semS.at[p]),
                pltpu.make_async_copy(rcp.at[pl.ds(st, C)], rcb.at[p],
                                      semS.at[p]),
                pltpu.make_async_copy(rvalp.at[pl.ds(st, C)], rvb.at[p],
                                      semS.at[p]),
            )

        def g1_copies(p):
            return (
                pltpu.make_async_copy(nrm_hbm.at[rcb.at[p]], ngb.at[p],
                                      semG.at[p]),
                pltpu.make_async_copy(rl_hbm.at[rcb.at[p]], rgb.at[p],
                                      semG.at[p]),
            )

        def g2_copies(p):
            return g1_copies(p) + (
                pltpu.make_async_copy(s_hbm.at[rowb.at[p]], sgb.at[p],
                                      semG.at[p]),
                pltpu.make_async_copy(dflat_hbm.at[didxb.at[p]], dbuf.at[p],
                                      semG.at[p]),
                pltpu.make_async_copy(feats_hbm.at[colb.at[p]], fbuf.at[p],
                                      semG.at[p]),
                pltpu.make_async_copy(rel_hbm.at[rcb.at[p]], rbuf.at[p],
                                      semG.at[p]),
            )

        def issue(cps):
            for cp in cps:
                cp.start()

        def drain(cps):
            for cp in cps:
                cp.wait()

        def scat_copy(p):
            return pltpu.make_async_copy(estage.at[p], s_sh.at[gidxb.at[p]],
                                         semA.at[p])

        def edge_scalars(i, p):
            out = []
            st = chunk_start(i)
            for g in range(ng):
                sl = pl.ds(g * LANES, LANES)
                t16 = st + g * LANES + lane
                rowv = rowb[p, sl]
                rvv = rvb[p, sl]
                valid = (t16 >= e_lo) & (t16 < e_hi)
                fv = rvv / jnp.maximum(rvv * ngb[p, sl], EPS)
                ev = jnp.where(valid, jnp.exp(fv * rgb[p, sl]), 0.0)
                lidx = jnp.clip(rowv - r0, 0, slot - 1)
                out.append((sl, fv, ev, lidx))
            return out

        @pl.when(nche > 0)
        def _p1_pro():
            issue(smalls_copies(0, 0))
            drain(smalls_copies(0, 0))
            issue(g1_copies(0))
            issue(smalls_copies(1, 1))

        def p1_body(i2, _):
            for pb in (0, 1):
                i = 2 * i2 + pb
                po = 1 - pb
                drain(smalls_copies(i + 1, po))
                issue(g1_copies(po))
                drain(g1_copies(pb))

                @pl.when(i2 >= 1)
                def _(pb=pb):
                    scat_copy(pb).wait()
                for (sl, fv, ev, lidx) in edge_scalars(i, pb):
                    estage[pb, sl] = ev
                    gidxb[pb, sl] = sidx * slot + lidx
                pltpu.async_copy(estage.at[pb], s_sh.at[gidxb.at[pb]],
                                 semA.at[pb], add=True)
                issue(smalls_copies(i + 2, pb))
            return 0
        lax.fori_loop(0, nche // 2, p1_body, 0)

        @pl.when(nche > 0)
        def _p1_epi():
            drain(g1_copies(0))
            drain(smalls_copies(nche + 1, 1))
            scat_copy(0).wait()
            scat_copy(1).wait()

        pltpu.sync_copy(s_sh.at[pl.ds(sbase, slot)], sloc)
        pltpu.sync_copy(sloc, s_hbm.at[pl.ds(r0, slot)])

        def didx_stage(p):
            for g in range(ng):
                sl = pl.ds(g * LANES, LANES)
                didxb[p, sl] = colb[p, sl] * R8 + rcb[p, sl]

        @pl.when(nche > 0)
        def _p2_pro():
            issue(smalls_copies(0, 0))
            drain(smalls_copies(0, 0))
            didx_stage(0)
            issue(g2_copies(0))
            issue(smalls_copies(1, 1))

        def p2_body(i2, _):
          for pb in (0, 1):
            i = 2 * i2 + pb
            po = 1 - pb
            drain(smalls_copies(i + 1, po))
            didx_stage(po)
            issue(g2_copies(po))
            drain(g2_copies(pb))

            for gi, (sl, fv, ev, lidx) in enumerate(edge_scalars(i, pb)):
                sgv = sgb[pb, sl]
                attn = ev / jnp.where(sgv > 0.0, sgv, 1.0)
                w2v = -2.0 * attn * fv * fv * dbuf[pb, sl]
                gbase = gi * LANES
                a_s = [attn[j] for j in range(LANES)]
                w_s = [w2v[j] for j in range(LANES)]
                l_s = [lidx[j] for j in range(LANES)]
                racc = [zf] * nk
                for jj in range(LANES):
                    e_row = gbase + jj
                    for k in range(nk):
                        ksl = pl.ds(k * LANES, LANES)
                        racc[k] = (racc[k] + a_s[jj] * fbuf[pb, e_row, ksl]
                                   + w_s[jj] * rbuf[pb, e_row, ksl])
                    if jj == LANES - 1:
                        for k in range(nk):
                            ksl = pl.ds(k * LANES, LANES)
                            acc[l_s[jj], ksl] = acc[l_s[jj], ksl] + racc[k]
                    else:
                        bnd = l_s[jj] != l_s[jj + 1]

                        @pl.when(bnd)
                        def _flush(jj=jj, racc=tuple(racc)):
                            for k in range(nk):
                                ksl = pl.ds(k * LANES, LANES)
                                acc[l_s[jj], ksl] = (acc[l_s[jj], ksl]
                                                     + racc[k])
                        racc = [jnp.where(bnd, 0.0, racc[k])
                                for k in range(nk)]
            issue(smalls_copies(i + 2, pb))
          return 0
        lax.fori_loop(0, nche // 2, p2_body, 0)

        @pl.when(nche > 0)
        def _p2_epi():
            drain(g2_copies(0))
            drain(smalls_copies(nche + 1, 1))

        def p3(i, _):
            for k in range(nk):
                ksl = pl.ds(k * LANES, LANES)
                x = acc[i, ksl]
                tt = jnp.exp(-2.0 * jnp.abs(x))
                acc[i, ksl] = jnp.sign(x) * (1.0 - tt) / (1.0 + tt)
            return 0
        lax.fori_loop(0, slot, p3, 0)
        pltpu.sync_copy(acc.at[pl.ds(0, rpt)], out_hbm.at[pl.ds(r0, rpt)])

    return pl.kernel(
        body,
        out_type=(jax.ShapeDtypeStruct((n_pad, d), jnp.float32),
                  jax.ShapeDtypeStruct((n_pad,), jnp.float32)),
        mesh=mesh,
        scratch_types=[
            pltpu.VMEM((LANES,), jnp.int32),
            pltpu.VMEM((2, C), jnp.int32),
            pltpu.VMEM((2, C), jnp.int32),
            pltpu.VMEM((2, C), jnp.int32),
            pltpu.VMEM((2, C), jnp.float32),
            pltpu.VMEM((2, C), jnp.float32),
            pltpu.VMEM((2, C), jnp.float32),
            pltpu.VMEM((2, C), jnp.float32),
            pltpu.VMEM((2, C), jnp.float32),
            pltpu.VMEM((2, C), jnp.float32),
            pltpu.VMEM((2, C), jnp.int32),
            pltpu.VMEM((2, C), jnp.int32),
            pltpu.VMEM((slot,), jnp.float32),
            pltpu.VMEM((2, C, d), jnp.float32),
            pltpu.VMEM((2, C, d), jnp.float32),
            pltpu.VMEM((slot, d), jnp.float32),
            pltpu.VMEM_SHARED((NS * slot,), jnp.float32),
            pltpu.SemaphoreType.DMA((2,)),
            pltpu.SemaphoreType.DMA((2,)),
            pltpu.SemaphoreType.DMA((2,)),
            pltpu.SemaphoreType.DMA,
        ],
        name="gat_sc_depth",
    )


def kernel(features, rel_emb, adj, r_index, r_val, triple_size, rel_size,
           node_size, attn_kernel_0, attn_kernel_1):
    n, d = features.shape
    t = r_val.shape[0]
    r = rel_emb.shape[0]
    rpt = -(-n // (NW * 8)) * 8
    slot = rpt
    n_pad = NW * rpt

    row = adj[0]
    col = adj[1]
    rc = r_index[1]
    t_pad = t + 8 * C

    def pad1(x, v):
        return jnp.concatenate([x, jnp.full((t_pad - t,), v, x.dtype)])

    rowp = pad1(row, jnp.int32(n))
    colp = pad1(col, jnp.int32(0))
    rcp = pad1(rc, jnp.int32(0))
    rvalp = pad1(r_val, jnp.float32(0))

    rel_pad = jnp.concatenate([rel_emb, jnp.zeros((R8 - r, d), jnp.float32)], 0)
    rel3 = rel_pad.reshape(8, 128, d)
    nrm3, rl03, rl13 = pl.pallas_call(
        _prep_rel_body,
        out_shape=[jax.ShapeDtypeStruct((8, 128), jnp.float32)] * 3,
    )(rel3, attn_kernel_0.reshape(1, 1, d), attn_kernel_1.reshape(1, 1, d))
    nrm = nrm3.reshape(R8)
    rl0 = rl03.reshape(R8)
    rl1 = rl13.reshape(R8)

    features_p = jnp.concatenate(
        [features, jnp.zeros((n_pad - n, d), jnp.float32)], 0)
    bm = 512
    feats0 = pl.pallas_call(
        _tanh_body,
        grid=(n_pad // bm,),
        in_specs=[pl.BlockSpec((bm, d), lambda i: (i, 0))],
        out_specs=pl.BlockSpec((bm, d), lambda i: (i, 0)),
        out_shape=jax.ShapeDtypeStruct((n_pad, d), jnp.float32),
    )(features_p)

    def dmat(feats):
        dm = pl.pallas_call(
            _dmat_body,
            grid=(n_pad // bm,),
            in_specs=[pl.BlockSpec((bm, d), lambda i: (i, 0)),
                      pl.BlockSpec((R8, d), lambda i: (0, 0))],
            out_specs=pl.BlockSpec((bm, R8), lambda i: (i, 0)),
            out_shape=jax.ShapeDtypeStruct((n_pad, R8), jnp.float32),
        )(feats, rel_pad)
        return dm.reshape(n_pad * R8)

    bounds = jnp.searchsorted(
        row, jnp.arange(NW + 1, dtype=jnp.int32) * rpt, side="left"
    ).astype(jnp.int32)
    e_lo = bounds[:NW]
    e_hi = bounds[1:]
    a0 = (e_lo // 8) * 8
    nch = (e_hi - a0 + C - 1) // C
    zc = jnp.zeros((NW,), jnp.int32)
    binfo = jnp.stack([e_lo, e_hi, a0, nch] + [zc] * 12, axis=1)

    zrows = jnp.zeros((slot, d), jnp.float32)
    sc = _make_sc_depth(d, rpt, slot, n_pad)
    f1p, _ = sc(feats0, rel_pad, dmat(feats0), rowp, colp, rcp, rvalp,
                nrm, rl0, binfo, zrows)
    f2p, _ = sc(f1p, rel_pad, dmat(f1p), rowp, colp, rcp, rvalp,
                nrm, rl1, binfo, zrows)

    return jnp.concatenate([feats0[:n], f1p[:n], f2p[:n]], axis=-1)


# speedup vs baseline: 1.1619x; 1.1619x over previous
import jax
import jax.numpy as jnp
from jax import lax
from jax.experimental import pallas as pl
from jax.experimental.pallas import tpu as pltpu
from jax.experimental.pallas import tpu_sc as plsc

NC, NS, LANES = 2, 16, 16
NW = NC * NS
C = 64
R8 = 1024
EPS = 1e-12


def _prep_rel_body(rel3_ref, ak0_ref, ak1_ref, nrm_ref, rl0_ref, rl1_ref):
    r = rel3_ref[...]
    nrm_ref[...] = jnp.sqrt(jnp.sum(r * r, axis=2))
    rl0_ref[...] = jnp.sum(r * ak0_ref[...], axis=2)
    rl1_ref[...] = jnp.sum(r * ak1_ref[...], axis=2)


def _tanh_body(x_ref, o_ref):
    o_ref[...] = jnp.tanh(x_ref[...])


def _dmat_body(f_ref, r_ref, o_ref):
    o_ref[...] = lax.dot_general(
        f_ref[...], r_ref[...], (((1,), (1,)), ((), ())),
        preferred_element_type=jnp.float32)


def _make_sc_depth(d, rpt, slot, n_pad):
    mesh = plsc.VectorSubcoreMesh(core_axis_name="c", subcore_axis_name="s",
                                  num_cores=NC, num_subcores=NS)

    def body(feats_hbm, rel_hbm, dflat_hbm, epack_hbm, rvalp, nrm_hbm,
             rl_hbm, binfo_hbm, zrows_hbm, out_hbm, s_hbm, bbuf, ebuf,
             rvb, ngb, rgb, sgb, dbuf, estage, gidxb, sloc, fbuf,
             rbuf, acc, s_sh, sem, sem2):
        cidx = lax.axis_index("c")
        sidx = lax.axis_index("s")
        w = cidx * NS + sidx
        lane = lax.iota(jnp.int32, LANES)

        pltpu.sync_copy(binfo_hbm.at[w], bbuf)
        bv = bbuf[...]
        e_lo, e_hi, a0c, nch = bv[0], bv[1], bv[2], bv[3]
        r0 = pl.multiple_of(w * rpt, 8)

        zf = jnp.zeros((LANES,), jnp.float32)

        cp_z = pltpu.async_copy(zrows_hbm, acc, sem2)
        for i in range(slot // LANES):
            sloc[pl.ds(i * LANES, LANES)] = zf
        sbase = pl.multiple_of(sidx * slot, 8)
        pltpu.sync_copy(sloc, s_sh.at[pl.ds(sbase, slot)])
        cp_z.wait()

        def p1_chunk(i, _):
            ci = a0c + i
            start = ci * C
            cv = pltpu.async_copy(rvalp.at[pl.ds(start, C)], rvb, sem2)
            pltpu.sync_copy(epack_hbm.at[ci], ebuf)
            c4 = pltpu.async_copy(nrm_hbm.at[ebuf.at[2]], ngb, sem)
            c5 = pltpu.async_copy(rl_hbm.at[ebuf.at[2]], rgb, sem)
            c4.wait(); c5.wait(); cv.wait()
            for g in range(C // LANES):
                sl = pl.ds(g * LANES, LANES)
                t16 = start + g * LANES + lane
                rowv = ebuf[0, sl]
                rvv = rvb[sl]
                valid = (t16 >= e_lo) & (t16 < e_hi)
                fv = rvv / jnp.maximum(rvv * ngb[sl], EPS)
                ev = jnp.where(valid, jnp.exp(fv * rgb[sl]), 0.0)
                lidx = jnp.clip(rowv - r0, 0, slot - 1)
                estage[sl] = ev
                gidxb[sl] = sidx * slot + lidx
            pltpu.sync_copy(estage, s_sh.at[gidxb], add=True)
            return 0
        lax.fori_loop(0, nch, p1_chunk, 0)

        pltpu.sync_copy(s_sh.at[pl.ds(sbase, slot)], sloc)
        pltpu.sync_copy(sloc, s_hbm.at[pl.ds(r0, slot)])

        def p2_chunk(i, _):
            ci = a0c + i
            start = ci * C
            cv = pltpu.async_copy(rvalp.at[pl.ds(start, C)], rvb, sem)
            pltpu.sync_copy(epack_hbm.at[ci], ebuf)
            g1 = pltpu.async_copy(nrm_hbm.at[ebuf.at[2]], ngb, sem)
            g2 = pltpu.async_copy(rl_hbm.at[ebuf.at[2]], rgb, sem)
            g3 = pltpu.async_copy(s_hbm.at[ebuf.at[0]], sgb, sem)
            g4 = pltpu.async_copy(dflat_hbm.at[ebuf.at[3]], dbuf, sem)
            g5 = pltpu.async_copy(feats_hbm.at[ebuf.at[1]], fbuf, sem2)
            g6 = pltpu.async_copy(rel_hbm.at[ebuf.at[2]], rbuf, sem2)
            g1.wait(); g2.wait(); g3.wait(); g4.wait(); g5.wait(); g6.wait(); cv.wait()

            def grp(g, _):
                sl = pl.ds(g * LANES, LANES)
                t16 = start + g * LANES + lane
                rowv = ebuf[0, sl]
                rvv = rvb[sl]
                valid = (t16 >= e_lo) & (t16 < e_hi)
                fv = rvv / jnp.maximum(rvv * ngb[sl], EPS)
                ev = jnp.where(valid, jnp.exp(fv * rgb[sl]), 0.0)
                sgv = sgb[sl]
                attn = ev / jnp.where(sgv > 0.0, sgv, 1.0)
                w2v = -2.0 * attn * fv * fv * dbuf[sl]
                lidx = jnp.clip(rowv - r0, 0, slot - 1)
                gbase = g * LANES
                a_s = [attn[j] for j in range(LANES)]
                w_s = [w2v[j] for j in range(LANES)]
                l_s = [lidx[j] for j in range(LANES)]
                nk = d // LANES
                racc = [zf] * nk
                for jj in range(LANES):
                    e_row = gbase + jj
                    for k in range(nk):
                        ksl = pl.ds(k * LANES, LANES)
                        racc[k] = (racc[k] + a_s[jj] * fbuf[e_row, ksl]
                                   + w_s[jj] * rbuf[e_row, ksl])
                    if jj == LANES - 1:
                        for k in range(nk):
                            ksl = pl.ds(k * LANES, LANES)
                            acc[l_s[jj], ksl] = acc[l_s[jj], ksl] + racc[k]
                    else:
                        bnd = l_s[jj] != l_s[jj + 1]

                        @pl.when(bnd)
                        def _flush(jj=jj, racc=tuple(racc)):
                            for k in range(nk):
                                ksl = pl.ds(k * LANES, LANES)
                                acc[l_s[jj], ksl] = acc[l_s[jj], ksl] + racc[k]
                        racc = [jnp.where(bnd, 0.0, racc[k])
                                for k in range(nk)]
                return 0
            lax.fori_loop(0, C // LANES, grp, 0)
            return 0
        lax.fori_loop(0, nch, p2_chunk, 0)

        def p3(i, _):
            for k in range(d // LANES):
                ksl = pl.ds(k * LANES, LANES)
                x = acc[i, ksl]
                tt = jnp.exp(-2.0 * jnp.abs(x))
                acc[i, ksl] = jnp.sign(x) * (1.0 - tt) / (1.0 + tt)
            return 0
        lax.fori_loop(0, slot, p3, 0)
        pltpu.sync_copy(acc.at[pl.ds(0, rpt)], out_hbm.at[pl.ds(r0, rpt)])

    return pl.kernel(
        body,
        out_type=(jax.ShapeDtypeStruct((n_pad, d), jnp.float32),
                  jax.ShapeDtypeStruct((n_pad,), jnp.float32)),
        mesh=mesh,
        scratch_types=[
            pltpu.VMEM((LANES,), jnp.int32),
            pltpu.VMEM((8, C), jnp.int32),
            pltpu.VMEM((C,), jnp.float32),
            pltpu.VMEM((C,), jnp.float32),
            pltpu.VMEM((C,), jnp.float32),
            pltpu.VMEM((C,), jnp.float32),
            pltpu.VMEM((C,), jnp.float32),
            pltpu.VMEM((C,), jnp.float32),
            pltpu.VMEM((C,), jnp.int32),
            pltpu.VMEM((slot,), jnp.float32),
            pltpu.VMEM((C, d), jnp.float32),
            pltpu.VMEM((C, d), jnp.float32),
            pltpu.VMEM((slot, d), jnp.float32),
            pltpu.VMEM_SHARED((NS * slot,), jnp.float32),
            pltpu.SemaphoreType.DMA,
            pltpu.SemaphoreType.DMA,
        ],
        name="gat_sc_depth",
    )


def kernel(features, rel_emb, adj, r_index, r_val, triple_size, rel_size,
           node_size, attn_kernel_0, attn_kernel_1):
    n, d = features.shape
    t = r_val.shape[0]
    r = rel_emb.shape[0]
    rpt = -(-n // (NW * 8)) * 8
    slot = rpt
    n_pad = NW * rpt

    row = adj[0]
    col = adj[1]
    rc = r_index[1]
    t_pad = t + 2 * C

    def pad1(x, v):
        return jnp.concatenate([x, jnp.full((t_pad - t,), v, x.dtype)])

    rowp = pad1(row, jnp.int32(n))
    colp = pad1(col, jnp.int32(0))
    rcp = pad1(rc, jnp.int32(0))
    rvalp = pad1(r_val, jnp.float32(0))
    didxp = colp * R8 + rcp
    zi = jnp.zeros((t_pad,), jnp.int32)
    epack = jnp.stack([rowp, colp, rcp, didxp, zi, zi, zi, zi], 0)
    epack = epack.reshape(8, t_pad // C, C).transpose(1, 0, 2)

    rel_pad = jnp.concatenate([rel_emb, jnp.zeros((R8 - r, d), jnp.float32)], 0)
    rel3 = rel_pad.reshape(8, 128, d)
    nrm3, rl03, rl13 = pl.pallas_call(
        _prep_rel_body,
        out_shape=[jax.ShapeDtypeStruct((8, 128), jnp.float32)] * 3,
    )(rel3, attn_kernel_0.reshape(1, 1, d), attn_kernel_1.reshape(1, 1, d))
    nrm = nrm3.reshape(R8)
    rl0 = rl03.reshape(R8)
    rl1 = rl13.reshape(R8)

    features_p = jnp.concatenate(
        [features, jnp.zeros((n_pad - n, d), jnp.float32)], 0)
    bm = 512
    feats0 = pl.pallas_call(
        _tanh_body,
        grid=(n_pad // bm,),
        in_specs=[pl.BlockSpec((bm, d), lambda i: (i, 0))],
        out_specs=pl.BlockSpec((bm, d), lambda i: (i, 0)),
        out_shape=jax.ShapeDtypeStruct((n_pad, d), jnp.float32),
    )(features_p)

    def dmat(feats):
        dm = pl.pallas_call(
            _dmat_body,
            grid=(n_pad // bm,),
            in_specs=[pl.BlockSpec((bm, d), lambda i: (i, 0)),
                      pl.BlockSpec((R8, d), lambda i: (0, 0))],
            out_specs=pl.BlockSpec((bm, R8), lambda i: (i, 0)),
            out_shape=jax.ShapeDtypeStruct((n_pad, R8), jnp.float32),
        )(feats, rel_pad)
        return dm.reshape(n_pad * R8)

    bounds = jnp.searchsorted(
        row, jnp.arange(NW + 1, dtype=jnp.int32) * rpt, side="left"
    ).astype(jnp.int32)
    e_lo = bounds[:NW]
    e_hi = bounds[1:]
    a0c = e_lo // C
    nch = (e_hi - a0c * C + C - 1) // C
    zc = jnp.zeros((NW,), jnp.int32)
    binfo = jnp.stack([e_lo, e_hi, a0c, nch] + [zc] * 12, axis=1)

    zrows = jnp.zeros((slot, d), jnp.float32)
    sc = _make_sc_depth(d, rpt, slot, n_pad)
    f1p, _ = sc(feats0, rel_pad, dmat(feats0), epack, rvalp, nrm, rl0,
                binfo, zrows)
    f2p, _ = sc(f1p, rel_pad, dmat(f1p), epack, rvalp, nrm, rl1,
                binfo, zrows)

    return jnp.concatenate([feats0[:n], f1p[:n], f2p[:n]], axis=-1)
